# Initial kernel scaffold; baseline (speedup 1.0000x reference)
import functools

import jax
import jax.numpy as jnp
from jax import lax
from jax.experimental import pallas as pl
from jax.experimental.pallas import tpu as pltpu
from jax.experimental.pallas import tpu_sc as plsc

B = 16384
D = 32
NS = 26
SV = 100000
NT = NS + 2
HID = 2 * D

NC = 2
NSUB = 16
NW = NC * NSUB
BPW = B // NW
CHUNK = 128
NCH = BPW // CHUNK


def _make_sc_gather():
    mesh = plsc.VectorSubcoreMesh(core_axis_name="c", subcore_axis_name="s")

    @functools.partial(
        pl.kernel,
        mesh=mesh,
        out_type=jax.ShapeDtypeStruct((NT, B, D), jnp.float32),
        scratch_types=[
            pltpu.VMEM((NCH, CHUNK), jnp.int32),
            pltpu.VMEM((BPW, D), jnp.float32),
            pltpu.SemaphoreType.DMA,
        ],
    )
    def sc_gather(user_hbm, item_hbm, sparse_hbm, idx_hbm, out_hbm,
                  idx_v, rows_v, sem):
        wid = lax.axis_index("s") * NC + lax.axis_index("c")
        base = wid * BPW

        def gather_one(table, t):
            pltpu.sync_copy(idx_hbm.at[t, wid], idx_v)
            cps = [
                pltpu.async_copy(
                    table.at[idx_v.at[c]],
                    rows_v.at[pl.ds(c * CHUNK, CHUNK)],
                    sem,
                )
                for c in range(NCH)
            ]
            for cp in cps:
                cp.wait()
            pltpu.sync_copy(rows_v, out_hbm.at[t, pl.ds(base, BPW)])

        gather_one(user_hbm, 0)
        gather_one(item_hbm, 1)

        def body(f, carry):
            gather_one(sparse_hbm, f + 2)
            return carry

        lax.fori_loop(0, NS, body, 0)

    return sc_gather


_sc_gather = _make_sc_gather()


BB = 512


def _mlp_body(g_ref, w1_ref, b1_ref, w2t_ref, b2_ref, out_ref):
    acc = jnp.broadcast_to(b1_ref[...], (BB, HID))
    for t in range(NT):
        acc = acc + jnp.dot(
            g_ref[t], w1_ref[t], preferred_element_type=jnp.float32
        )
    h = jnp.maximum(acc, 0.0)
    raw = jnp.sum(h * w2t_ref[...], axis=1, keepdims=True) + b2_ref[...]
    out_ref[...] = jax.nn.sigmoid(raw)


def _mlp(gathered, w1r, b1r, w2t, b2r):
    grid = (B // BB,)
    return pl.pallas_call(
        _mlp_body,
        grid=grid,
        in_specs=[
            pl.BlockSpec((NT, BB, D), lambda i: (0, i, 0)),
            pl.BlockSpec((NT, D, HID), lambda i: (0, 0, 0)),
            pl.BlockSpec((1, HID), lambda i: (0, 0)),
            pl.BlockSpec((1, HID), lambda i: (0, 0)),
            pl.BlockSpec((1, 1), lambda i: (0, 0)),
        ],
        out_specs=pl.BlockSpec((BB, 1), lambda i: (i, 0)),
        out_shape=jax.ShapeDtypeStruct((B, 1), jnp.float32),
    )(gathered, w1r, b1r, w2t, b2r)


def kernel(user_ids, item_ids, sparse_features, user_table, item_table,
           sparse_tables, W1, b1, W2, b2):
    sf = sparse_features.astype(jnp.int32) + (
        jnp.arange(NS, dtype=jnp.int32) * SV
    )[None, :]
    idx_all = jnp.concatenate(
        [user_ids.astype(jnp.int32)[:, None],
         item_ids.astype(jnp.int32)[:, None],
         sf],
        axis=1,
    )
    idx_all = idx_all.T.reshape(NT, NW, NCH, CHUNK)
    flat_sparse = sparse_tables.reshape(NS * SV, D)

    gathered = _sc_gather(user_table, item_table, flat_sparse, idx_all)

    w1r = W1.reshape(NT, D, HID)
    b1r = b1.reshape(1, HID)
    w2t = W2.reshape(1, HID)
    b2r = b2.reshape(1, 1)
    return _mlp(gathered, w1r, b1r, w2t, b2r)


# speedup vs baseline: 5.1706x; 5.1706x over previous
import functools

import jax
import jax.numpy as jnp
from jax import lax
from jax.experimental import pallas as pl
from jax.experimental.pallas import tpu as pltpu
from jax.experimental.pallas import tpu_sc as plsc

B = 16384
D = 32
NS = 26
SV = 100000
NT = NS + 2
HID = 2 * D

NC = 2
NSUB = 16
NW = NC * NSUB
BPW = B // NW
CHUNK = 128
NCH = BPW // CHUNK


def _make_sc_gather():
    mesh = plsc.VectorSubcoreMesh(core_axis_name="c", subcore_axis_name="s")

    @functools.partial(
        pl.kernel,
        mesh=mesh,
        out_type=jax.ShapeDtypeStruct((NT, B, D), jnp.float32),
        scratch_types=[
            pltpu.VMEM((NCH, CHUNK), jnp.int32),
            pltpu.VMEM((BPW, D), jnp.float32),
            pltpu.SemaphoreType.DMA,
        ],
        compiler_params=pltpu.CompilerParams(use_tc_tiling_on_sc=False),
    )
    def sc_gather(user_hbm, item_hbm, sparse_hbm, idx_hbm, out_hbm,
                  idx_v, rows_v, sem):
        wid = lax.axis_index("s") * NC + lax.axis_index("c")
        base = wid * BPW

        def gather_one(table, t):
            pltpu.sync_copy(idx_hbm.at[t, wid], idx_v)
            cps = [
                pltpu.async_copy(
                    table.at[idx_v.at[c]],
                    rows_v.at[pl.ds(c * CHUNK, CHUNK)],
                    sem,
                )
                for c in range(NCH)
            ]
            for cp in cps:
                cp.wait()
            pltpu.sync_copy(rows_v, out_hbm.at[t, pl.ds(base, BPW)])

        gather_one(user_hbm, 0)
        gather_one(item_hbm, 1)

        def body(f, carry):
            gather_one(sparse_hbm, f + 2)
            return carry

        lax.fori_loop(0, NS, body, 0)

    return sc_gather


_sc_gather = _make_sc_gather()


BB = 512


def _mlp_body(g_ref, w1_ref, b1_ref, w2t_ref, b2_ref, out_ref):
    acc = jnp.broadcast_to(b1_ref[...], (BB, HID))
    for t in range(NT):
        acc = acc + jnp.dot(
            g_ref[t], w1_ref[t], preferred_element_type=jnp.float32
        )
    h = jnp.maximum(acc, 0.0)
    raw = jnp.sum(h * w2t_ref[...], axis=1, keepdims=True) + b2_ref[...]
    out_ref[...] = jax.nn.sigmoid(raw)


def _mlp(gathered, w1r, b1r, w2t, b2r):
    grid = (B // BB,)
    return pl.pallas_call(
        _mlp_body,
        grid=grid,
        in_specs=[
            pl.BlockSpec((NT, BB, D), lambda i: (0, i, 0)),
            pl.BlockSpec((NT, D, HID), lambda i: (0, 0, 0)),
            pl.BlockSpec((1, HID), lambda i: (0, 0)),
            pl.BlockSpec((1, HID), lambda i: (0, 0)),
            pl.BlockSpec((1, 1), lambda i: (0, 0)),
        ],
        out_specs=pl.BlockSpec((BB, 1), lambda i: (i, 0)),
        out_shape=jax.ShapeDtypeStruct((B, 1), jnp.float32),
    )(gathered, w1r, b1r, w2t, b2r)


def kernel(user_ids, item_ids, sparse_features, user_table, item_table,
           sparse_tables, W1, b1, W2, b2):
    sf = sparse_features.astype(jnp.int32) + (
        jnp.arange(NS, dtype=jnp.int32) * SV
    )[None, :]
    idx_all = jnp.concatenate(
        [user_ids.astype(jnp.int32)[:, None],
         item_ids.astype(jnp.int32)[:, None],
         sf],
        axis=1,
    )
    idx_all = idx_all.T.reshape(NT, NW, NCH, CHUNK)
    flat_sparse = sparse_tables.reshape(NS * SV, D)

    gathered = _sc_gather(user_table, item_table, flat_sparse, idx_all)

    w1r = W1.reshape(NT, D, HID)
    b1r = b1.reshape(1, HID)
    w2t = W2.reshape(1, HID)
    b2r = b2.reshape(1, 1)
    return _mlp(gathered, w1r, b1r, w2t, b2r)


# speedup vs baseline: 5.6537x; 1.0934x over previous
import functools

import jax
import jax.numpy as jnp
from jax import lax
from jax.experimental import pallas as pl
from jax.experimental.pallas import tpu as pltpu
from jax.experimental.pallas import tpu_sc as plsc

B = 16384
D = 32
NS = 26
SV = 100000
NT = NS + 2
HID = 2 * D

NC = 2
NSUB = 16
NW = NC * NSUB
BPW = B // NW
CHUNK = 128
NCH = BPW // CHUNK


def _make_sc_gather():
    mesh = plsc.VectorSubcoreMesh(core_axis_name="c", subcore_axis_name="s")

    @functools.partial(
        pl.kernel,
        mesh=mesh,
        out_type=jax.ShapeDtypeStruct((NT, B, D), jnp.float32),
        scratch_types=[
            pltpu.VMEM((NCH, CHUNK), jnp.int32),
            pltpu.VMEM((BPW, D), jnp.float32),
            pltpu.SemaphoreType.DMA,
        ],
        compiler_params=pltpu.CompilerParams(use_tc_tiling_on_sc=False),
    )
    def sc_gather(user_hbm, item_hbm, sparse_hbm, idx_hbm, out_hbm,
                  idx_v, rows_v, sem):
        wid = lax.axis_index("s") * NC + lax.axis_index("c")
        base = wid * BPW

        def gather_one(table, t):
            pltpu.sync_copy(idx_hbm.at[t, wid], idx_v)
            cps = [
                pltpu.async_copy(
                    table.at[idx_v.at[c]],
                    rows_v.at[pl.ds(c * CHUNK, CHUNK)],
                    sem,
                )
                for c in range(NCH)
            ]
            for cp in cps:
                cp.wait()
            pltpu.sync_copy(rows_v, out_hbm.at[t, pl.ds(base, BPW)])

        gather_one(user_hbm, 0)
        gather_one(item_hbm, 1)

        def body(f, carry):
            gather_one(sparse_hbm, f + 2)
            return carry

        lax.fori_loop(0, NS, body, 0)

    return sc_gather


_sc_gather = _make_sc_gather()


PK = 128 // D
RB = 128
GR = B * D // 128


def _mlp_body(g_ref, w1e_ref, b1t_ref, w2e_ref, b2e_ref, out_ref):
    acc = jnp.broadcast_to(b1t_ref[...], (RB, PK * HID))
    for t in range(NT):
        acc = acc + jnp.dot(
            g_ref[t], w1e_ref[t], preferred_element_type=jnp.float32
        )
    h = jnp.maximum(acc, 0.0)
    raw = jnp.dot(h, w2e_ref[...], preferred_element_type=jnp.float32)
    out_ref[...] = jax.nn.sigmoid(raw + b2e_ref[...])


def _mlp(gathered, w1e, b1t, w2e, b2e):
    grid = (GR // RB,)
    return pl.pallas_call(
        _mlp_body,
        grid=grid,
        in_specs=[
            pl.BlockSpec((NT, RB, 128), lambda i: (0, i, 0)),
            pl.BlockSpec((NT, 128, PK * HID), lambda i: (0, 0, 0)),
            pl.BlockSpec((1, PK * HID), lambda i: (0, 0)),
            pl.BlockSpec((PK * HID, PK), lambda i: (0, 0)),
            pl.BlockSpec((1, PK), lambda i: (0, 0)),
        ],
        out_specs=pl.BlockSpec((RB, PK), lambda i: (i, 0)),
        out_shape=jax.ShapeDtypeStruct((GR, PK), jnp.float32),
    )(gathered, w1e, b1t, w2e, b2e)


def kernel(user_ids, item_ids, sparse_features, user_table, item_table,
           sparse_tables, W1, b1, W2, b2):
    sf = sparse_features.astype(jnp.int32) + (
        jnp.arange(NS, dtype=jnp.int32) * SV
    )[None, :]
    idx_all = jnp.concatenate(
        [user_ids.astype(jnp.int32)[:, None],
         item_ids.astype(jnp.int32)[:, None],
         sf],
        axis=1,
    )
    idx_all = idx_all.T.reshape(NT, NW, NCH, CHUNK)
    flat_sparse = sparse_tables.reshape(NS * SV, D)

    gathered = _sc_gather(user_table, item_table, flat_sparse, idx_all)
    gathered = gathered.reshape(NT, GR, 128)

    w1r = W1.reshape(NT, D, HID)
    w1e = jnp.zeros((NT, 128, PK * HID), jnp.float32)
    w2e = jnp.zeros((PK * HID, PK), jnp.float32)
    for k in range(PK):
        w1e = w1e.at[:, D * k:D * (k + 1), HID * k:HID * (k + 1)].set(w1r)
        w2e = w2e.at[HID * k:HID * (k + 1), k].set(W2[:, 0])
    b1t = jnp.tile(b1, PK).reshape(1, PK * HID)
    b2e = jnp.broadcast_to(b2.reshape(1, 1), (1, PK))

    out = _mlp(gathered, w1e, b1t, w2e, b2e)
    return out.reshape(B, 1)


# speedup vs baseline: 5.8467x; 1.0341x over previous
import functools

import jax
import jax.numpy as jnp
from jax import lax
from jax.experimental import pallas as pl
from jax.experimental.pallas import tpu as pltpu
from jax.experimental.pallas import tpu_sc as plsc

B = 16384
D = 32
NS = 26
SV = 100000
NT = NS + 2
HID = 2 * D

NC = 2
NSUB = 16
NW = NC * NSUB
BPW = B // NW
CHUNK = 128
NCH = BPW // CHUNK
NQ = NT * NCH

PK = 128 // D
GR = B * D // 128

UV = 1000000
BN = 2048
BLK = PK * BN
NBU = -(-UV // BLK)
NBS = -(-SV // BLK)
FS = NBS * BN


def _tpack2_body(x0, x1, x2, x3, o):
    o[...] = jnp.concatenate(
        [x0[...].T, x1[...].T, x2[...].T, x3[...].T], axis=1
    )


def _tpack2(tT):
    last = (UV - 1) // BN
    in_specs = [
        pl.BlockSpec((D, BN), lambda i, a=a: (0, jnp.minimum(PK * i + a, last)))
        for a in range(PK)
    ]
    return pl.pallas_call(
        _tpack2_body,
        grid=(NBU,),
        in_specs=in_specs,
        out_specs=pl.BlockSpec((BN, 128), lambda i: (i, 0)),
        out_shape=jax.ShapeDtypeStruct((NBU * BN, 128), jnp.float32),
    )(tT, tT, tT, tT)


def _tpack3_body(x0, x1, x2, x3, o):
    o[...] = jnp.concatenate(
        [x0[0].T, x1[0].T, x2[0].T, x3[0].T], axis=1
    )


def _tpack3(tT3):
    last = (SV - 1) // BN
    in_specs = [
        pl.BlockSpec(
            (1, D, BN),
            lambda f, i, a=a: (f, 0, jnp.minimum(PK * i + a, last)),
        )
        for a in range(PK)
    ]
    return pl.pallas_call(
        _tpack3_body,
        grid=(NS, NBS),
        in_specs=in_specs,
        out_specs=pl.BlockSpec((BN, 128), lambda f, i: (f * NBS + i, 0)),
        out_shape=jax.ShapeDtypeStruct((NS * FS, 128), jnp.float32),
    )(tT3, tT3, tT3, tT3)


def _make_sc_gather():
    mesh = plsc.VectorSubcoreMesh(core_axis_name="c", subcore_axis_name="s")

    @functools.partial(
        pl.kernel,
        mesh=mesh,
        out_type=jax.ShapeDtypeStruct((NT, GR, 128), jnp.float32),
        scratch_types=[
            pltpu.VMEM((NQ, 128), jnp.int32),
            pltpu.VMEM((NQ, 128), jnp.int32),
            pltpu.VMEM((BPW, 128), jnp.float32),
            pltpu.VMEM((128, 128), jnp.float32),
            pltpu.SemaphoreType.DMA,
        ],
        compiler_params=pltpu.CompilerParams(
            use_tc_tiling_on_sc=False, needs_layout_passes=False
        ),
    )
    def sc_gather(user_hbm, item_hbm, sparse_hbm, iq_hbm, im_hbm, out_hbm,
                  iqv, imv, rows_v, out_v, sem):
        wid = lax.axis_index("s") * NC + lax.axis_index("c")
        pltpu.sync_copy(iq_hbm.at[wid], iqv)
        pltpu.sync_copy(im_hbm.at[wid], imv)
        iota16 = lax.iota(jnp.int32, 16)

        def do_table(table, t):
            cps = [
                pltpu.async_copy(
                    table.at[iqv.at[t * NCH + c]],
                    rows_v.at[pl.ds(c * CHUNK, CHUNK)],
                    sem,
                )
                for c in range(NCH)
            ]
            for cp in cps:
                cp.wait()

            def grp_body(g, carry):
                r0 = g * 16
                m16 = imv.at[t * NCH + (r0 // CHUNK),
                             pl.ds(r0 % CHUNK, 16)][...]
                rowids = r0 + iota16
                flatbase = rowids * D
                for d in range(D):
                    vals = plsc.load_gather(rows_v, [rowids, m16 + d])
                    flat = flatbase + d
                    plsc.store_scatter(
                        out_v, [flat // 128, flat % 128], vals
                    )
                return carry

            lax.fori_loop(0, BPW // 16, grp_body, 0)
            pltpu.sync_copy(out_v, out_hbm.at[t, pl.ds(wid * 128, 128)])

        do_table(user_hbm, 0)
        do_table(item_hbm, 1)

        def body(f, carry):
            do_table(sparse_hbm, f + 2)
            return carry

        lax.fori_loop(0, NS, body, 0)

    return sc_gather


_sc_gather = _make_sc_gather()


RB = 128


def _mlp_body(g_ref, w1e_ref, b1t_ref, w2e_ref, b2e_ref, out_ref):
    acc = jnp.broadcast_to(b1t_ref[...], (RB, PK * HID))
    for t in range(NT):
        acc = acc + jnp.dot(
            g_ref[t], w1e_ref[t], preferred_element_type=jnp.float32
        )
    h = jnp.maximum(acc, 0.0)
    raw = jnp.dot(h, w2e_ref[...], preferred_element_type=jnp.float32)
    out_ref[...] = jax.nn.sigmoid(raw + b2e_ref[...])


def _mlp(gathered, w1e, b1t, w2e, b2e):
    grid = (GR // RB,)
    return pl.pallas_call(
        _mlp_body,
        grid=grid,
        in_specs=[
            pl.BlockSpec((NT, RB, 128), lambda i: (0, i, 0)),
            pl.BlockSpec((NT, 128, PK * HID), lambda i: (0, 0, 0)),
            pl.BlockSpec((1, PK * HID), lambda i: (0, 0)),
            pl.BlockSpec((PK * HID, PK), lambda i: (0, 0)),
            pl.BlockSpec((1, PK), lambda i: (0, 0)),
        ],
        out_specs=pl.BlockSpec((RB, PK), lambda i: (i, 0)),
        out_shape=jax.ShapeDtypeStruct((GR, PK), jnp.float32),
    )(gathered, w1e, b1t, w2e, b2e)


def kernel(user_ids, item_ids, sparse_features, user_table, item_table,
           sparse_tables, W1, b1, W2, b2):
    uids = user_ids.astype(jnp.int32)
    tids = item_ids.astype(jnp.int32)
    sf = sparse_features.astype(jnp.int32)

    def _pack_idx(v):
        return (v // BLK) * BN + v % BN, ((v % BLK) // BN) * D

    iqu, imu = _pack_idx(uids)
    iqi, imi = _pack_idx(tids)
    iqs, ims = _pack_idx(sf)
    iqs = iqs + (jnp.arange(NS, dtype=jnp.int32) * FS)[None, :]
    iq_all = jnp.concatenate([iqu[:, None], iqi[:, None], iqs], axis=1)
    im_all = jnp.concatenate([imu[:, None], imi[:, None], ims], axis=1)

    def _worker_layout(a):
        a = a.reshape(NW, NCH, CHUNK, NT).transpose(0, 3, 1, 2)
        return a.reshape(NW, NQ, CHUNK)

    iq = _worker_layout(iq_all)
    im = _worker_layout(im_all)

    u4 = _tpack2(user_table.T)
    i4 = _tpack2(item_table.T)
    s4 = _tpack3(jnp.swapaxes(sparse_tables, 1, 2))

    gathered = _sc_gather(u4, i4, s4, iq, im)

    w1r = W1.reshape(NT, D, HID)
    w1e = jnp.zeros((NT, 128, PK * HID), jnp.float32)
    w2e = jnp.zeros((PK * HID, PK), jnp.float32)
    for k in range(PK):
        w1e = w1e.at[:, D * k:D * (k + 1), HID * k:HID * (k + 1)].set(w1r)
        w2e = w2e.at[HID * k:HID * (k + 1), k].set(W2[:, 0])
    b1t = jnp.tile(b1, PK).reshape(1, PK * HID)
    b2e = jnp.broadcast_to(b2.reshape(1, 1), (1, PK))

    out = _mlp(gathered, w1e, b1t, w2e, b2e)
    return out.reshape(B, 1)


# speedup vs baseline: 7.6337x; 1.3056x over previous
import functools

import jax
import jax.numpy as jnp
from jax import lax
from jax.experimental import pallas as pl
from jax.experimental.pallas import tpu as pltpu
from jax.experimental.pallas import tpu_sc as plsc

B = 16384
D = 32
NS = 26
SV = 100000
NT = NS + 2
HID = 2 * D

NC = 2
NSUB = 16
NW = NC * NSUB
BPW = B // NW
CHUNK = 128
NCH = BPW // CHUNK
NQ = NT * NCH

PK = 128 // D
GR = B * D // 128

UV = 1000000
BN = 2048
BLK = PK * BN
NBU = -(-UV // BLK)
NBS = -(-SV // BLK)
FS = NBS * BN


def _mxu_t(x):
    return jax.lax.dot_general(
        x, jnp.eye(D, dtype=jnp.float32),
        ((( 0,), (0,)), ((), ())),
        preferred_element_type=jnp.float32,
    )


def _tpack2_body(x0, x1, x2, x3, o):
    o[...] = jnp.concatenate(
        [_mxu_t(x0[...]), _mxu_t(x1[...]), _mxu_t(x2[...]), _mxu_t(x3[...])],
        axis=1,
    )


def _tpack2(tT):
    last = (UV - 1) // BN
    in_specs = [
        pl.BlockSpec((D, BN), lambda i, a=a: (0, jnp.minimum(PK * i + a, last)))
        for a in range(PK)
    ]
    return pl.pallas_call(
        _tpack2_body,
        grid=(NBU,),
        in_specs=in_specs,
        out_specs=pl.BlockSpec((BN, 128), lambda i: (i, 0)),
        out_shape=jax.ShapeDtypeStruct((NBU * BN, 128), jnp.float32),
    )(tT, tT, tT, tT)


def _tpack3_body(x0, x1, x2, x3, o):
    o[...] = jnp.concatenate(
        [_mxu_t(x0[0]), _mxu_t(x1[0]), _mxu_t(x2[0]), _mxu_t(x3[0])],
        axis=1,
    )


def _tpack3(tT3):
    last = (SV - 1) // BN
    in_specs = [
        pl.BlockSpec(
            (1, D, BN),
            lambda f, i, a=a: (f, 0, jnp.minimum(PK * i + a, last)),
        )
        for a in range(PK)
    ]
    return pl.pallas_call(
        _tpack3_body,
        grid=(NS, NBS),
        in_specs=in_specs,
        out_specs=pl.BlockSpec((BN, 128), lambda f, i: (f * NBS + i, 0)),
        out_shape=jax.ShapeDtypeStruct((NS * FS, 128), jnp.float32),
    )(tT3, tT3, tT3, tT3)


def _make_sc_gather():
    mesh = plsc.VectorSubcoreMesh(core_axis_name="c", subcore_axis_name="s")

    @functools.partial(
        pl.kernel,
        mesh=mesh,
        out_type=jax.ShapeDtypeStruct((NT, GR, 128), jnp.float32),
        scratch_types=[
            pltpu.VMEM((NQ, 128), jnp.int32),
            pltpu.VMEM((NQ, 128), jnp.int32),
            pltpu.VMEM((BPW, 128), jnp.float32),
            pltpu.VMEM((128, 128), jnp.float32),
            pltpu.SemaphoreType.DMA,
        ],
        compiler_params=pltpu.CompilerParams(
            use_tc_tiling_on_sc=False, needs_layout_passes=False
        ),
    )
    def sc_gather(user_hbm, item_hbm, sparse_hbm, iq_hbm, im_hbm, out_hbm,
                  iqv, imv, rows_v, out_v, sem):
        wid = lax.axis_index("s") * NC + lax.axis_index("c")
        pltpu.sync_copy(iq_hbm.at[wid], iqv)
        pltpu.sync_copy(im_hbm.at[wid], imv)
        iota16 = lax.iota(jnp.int32, 16)

        def do_table(table, t):
            cps = [
                pltpu.async_copy(
                    table.at[iqv.at[t * NCH + c]],
                    rows_v.at[pl.ds(c * CHUNK, CHUNK)],
                    sem,
                )
                for c in range(NCH)
            ]
            for cp in cps:
                cp.wait()

            def grp_body(g, carry):
                r0 = g * 16
                m16 = imv.at[t * NCH + (r0 // CHUNK),
                             pl.ds(r0 % CHUNK, 16)][...]
                q0 = g * 4
                for j in range(16):
                    m = m16[j]
                    lo = rows_v.at[r0 + j, pl.ds(m, 16)][...]
                    hi = rows_v.at[r0 + j, pl.ds(m + 16, 16)][...]
                    out_v[q0 + j // 4, pl.ds((j % 4) * D, 16)] = lo
                    out_v[q0 + j // 4, pl.ds((j % 4) * D + 16, 16)] = hi
                return carry

            lax.fori_loop(0, BPW // 16, grp_body, 0)
            pltpu.sync_copy(out_v, out_hbm.at[t, pl.ds(wid * 128, 128)])

        do_table(user_hbm, 0)
        do_table(item_hbm, 1)

        def body(f, carry):
            do_table(sparse_hbm, f + 2)
            return carry

        lax.fori_loop(0, NS, body, 0)

    return sc_gather


_sc_gather = _make_sc_gather()


RB = 128


def _mlp_body(g_ref, w1e_ref, b1t_ref, w2e_ref, b2e_ref, out_ref):
    acc = jnp.broadcast_to(b1t_ref[...], (RB, PK * HID))
    for t in range(NT):
        acc = acc + jnp.dot(
            g_ref[t], w1e_ref[t], preferred_element_type=jnp.float32
        )
    h = jnp.maximum(acc, 0.0)
    raw = jnp.dot(h, w2e_ref[...], preferred_element_type=jnp.float32)
    out_ref[...] = jax.nn.sigmoid(raw + b2e_ref[...])


def _mlp(gathered, w1e, b1t, w2e, b2e):
    grid = (GR // RB,)
    return pl.pallas_call(
        _mlp_body,
        grid=grid,
        in_specs=[
            pl.BlockSpec((NT, RB, 128), lambda i: (0, i, 0)),
            pl.BlockSpec((NT, 128, PK * HID), lambda i: (0, 0, 0)),
            pl.BlockSpec((1, PK * HID), lambda i: (0, 0)),
            pl.BlockSpec((PK * HID, PK), lambda i: (0, 0)),
            pl.BlockSpec((1, PK), lambda i: (0, 0)),
        ],
        out_specs=pl.BlockSpec((RB, PK), lambda i: (i, 0)),
        out_shape=jax.ShapeDtypeStruct((GR, PK), jnp.float32),
    )(gathered, w1e, b1t, w2e, b2e)


def kernel(user_ids, item_ids, sparse_features, user_table, item_table,
           sparse_tables, W1, b1, W2, b2):
    uids = user_ids.astype(jnp.int32)
    tids = item_ids.astype(jnp.int32)
    sf = sparse_features.astype(jnp.int32)

    def _pack_idx(v):
        return (v // BLK) * BN + v % BN, ((v % BLK) // BN) * D

    iqu, imu = _pack_idx(uids)
    iqi, imi = _pack_idx(tids)
    iqs, ims = _pack_idx(sf)
    iqs = iqs + (jnp.arange(NS, dtype=jnp.int32) * FS)[None, :]
    iq_all = jnp.concatenate([iqu[:, None], iqi[:, None], iqs], axis=1)
    im_all = jnp.concatenate([imu[:, None], imi[:, None], ims], axis=1)

    def _worker_layout(a):
        a = a.reshape(NW, NCH, CHUNK, NT).transpose(0, 3, 1, 2)
        return a.reshape(NW, NQ, CHUNK)

    iq = _worker_layout(iq_all)
    im = _worker_layout(im_all)

    u4 = _tpack2(user_table.T)
    i4 = _tpack2(item_table.T)
    s4 = _tpack3(jnp.swapaxes(sparse_tables, 1, 2))

    gathered = _sc_gather(u4, i4, s4, iq, im)

    w1r = W1.reshape(NT, D, HID)
    w1e = jnp.zeros((NT, 128, PK * HID), jnp.float32)
    w2e = jnp.zeros((PK * HID, PK), jnp.float32)
    for k in range(PK):
        w1e = w1e.at[:, D * k:D * (k + 1), HID * k:HID * (k + 1)].set(w1r)
        w2e = w2e.at[HID * k:HID * (k + 1), k].set(W2[:, 0])
    b1t = jnp.tile(b1, PK).reshape(1, PK * HID)
    b2e = jnp.broadcast_to(b2.reshape(1, 1), (1, PK))

    out = _mlp(gathered, w1e, b1t, w2e, b2e)
    return out.reshape(B, 1)


# speedup vs baseline: 9.7520x; 1.2775x over previous
import functools

import jax
import jax.numpy as jnp
from jax import lax
from jax.experimental import pallas as pl
from jax.experimental.pallas import tpu as pltpu
from jax.experimental.pallas import tpu_sc as plsc

B = 16384
D = 32
NS = 26
SV = 100000
NT = NS + 2
HID = 2 * D

NC = 2
NSUB = 16
NW = NC * NSUB
BPW = B // NW
CHUNK = 128
NCH = BPW // CHUNK
NQ = NT * NCH

PK = 128 // D
GR = B * D // 128

UV = 1000000
BN = 2048
BLK = PK * BN
NBU = -(-UV // BLK)
NBS = -(-SV // BLK)
FS = NBS * BN


def _pack_t(xs):
    e = jnp.eye(128, dtype=jnp.float32)
    acc = None
    for a, x in enumerate(xs):
        y = jax.lax.dot_general(
            x, e[D * a:D * (a + 1), :],
            (((0,), (0,)), ((), ())),
            preferred_element_type=jnp.float32,
        )
        acc = y if acc is None else acc + y
    return acc


def _tpack2_body(x0, x1, x2, x3, o):
    o[...] = _pack_t([x0[...], x1[...], x2[...], x3[...]])


def _tpack2(tT):
    last = (UV - 1) // BN
    in_specs = [
        pl.BlockSpec((D, BN), lambda i, a=a: (0, jnp.minimum(PK * i + a, last)))
        for a in range(PK)
    ]
    return pl.pallas_call(
        _tpack2_body,
        grid=(NBU,),
        in_specs=in_specs,
        out_specs=pl.BlockSpec((BN, 128), lambda i: (i, 0)),
        out_shape=jax.ShapeDtypeStruct((NBU * BN, 128), jnp.float32),
    )(tT, tT, tT, tT)


def _tpack3_body(x0, x1, x2, x3, o):
    o[...] = _pack_t([x0[0], x1[0], x2[0], x3[0]])


def _tpack3(tT3):
    last = (SV - 1) // BN
    in_specs = [
        pl.BlockSpec(
            (1, D, BN),
            lambda f, i, a=a: (f, 0, jnp.minimum(PK * i + a, last)),
        )
        for a in range(PK)
    ]
    return pl.pallas_call(
        _tpack3_body,
        grid=(NS, NBS),
        in_specs=in_specs,
        out_specs=pl.BlockSpec((BN, 128), lambda f, i: (f * NBS + i, 0)),
        out_shape=jax.ShapeDtypeStruct((NS * FS, 128), jnp.float32),
    )(tT3, tT3, tT3, tT3)


def _make_sc_gather():
    mesh = plsc.VectorSubcoreMesh(core_axis_name="c", subcore_axis_name="s")

    @functools.partial(
        pl.kernel,
        mesh=mesh,
        out_type=jax.ShapeDtypeStruct((NT, GR, 128), jnp.float32),
        scratch_types=[
            pltpu.VMEM((NQ, 128), jnp.int32),
            pltpu.VMEM((NQ, 128), jnp.int32),
            pltpu.VMEM((BPW, 128), jnp.float32),
            pltpu.VMEM((128, 128), jnp.float32),
            pltpu.SemaphoreType.DMA,
        ],
        compiler_params=pltpu.CompilerParams(
            use_tc_tiling_on_sc=False, needs_layout_passes=False
        ),
    )
    def sc_gather(user_hbm, item_hbm, sparse_hbm, iq_hbm, im_hbm, out_hbm,
                  iqv, imv, rows_v, out_v, sem):
        wid = lax.axis_index("s") * NC + lax.axis_index("c")
        pltpu.sync_copy(iq_hbm.at[wid], iqv)
        pltpu.sync_copy(im_hbm.at[wid], imv)
        iota16 = lax.iota(jnp.int32, 16)

        def do_table(table, t):
            cps = [
                pltpu.async_copy(
                    table.at[iqv.at[t * NCH + c]],
                    rows_v.at[pl.ds(c * CHUNK, CHUNK)],
                    sem,
                )
                for c in range(NCH)
            ]
            for cp in cps:
                cp.wait()

            def grp_body(g, carry):
                r0 = g * 16
                m16 = imv.at[t * NCH + (r0 // CHUNK),
                             pl.ds(r0 % CHUNK, 16)][...]
                q0 = g * 4
                for j in range(16):
                    m = m16[j]
                    lo = rows_v.at[r0 + j, pl.ds(m, 16)][...]
                    hi = rows_v.at[r0 + j, pl.ds(m + 16, 16)][...]
                    out_v[q0 + j // 4, pl.ds((j % 4) * D, 16)] = lo
                    out_v[q0 + j // 4, pl.ds((j % 4) * D + 16, 16)] = hi
                return carry

            lax.fori_loop(0, BPW // 16, grp_body, 0)
            pltpu.sync_copy(out_v, out_hbm.at[t, pl.ds(wid * 128, 128)])

        do_table(user_hbm, 0)
        do_table(item_hbm, 1)

        def body(f, carry):
            do_table(sparse_hbm, f + 2)
            return carry

        lax.fori_loop(0, NS, body, 0)

    return sc_gather


_sc_gather = _make_sc_gather()


RB = 128


def _mlp_body(g_ref, w1e_ref, b1t_ref, w2e_ref, b2e_ref, out_ref):
    acc = jnp.broadcast_to(b1t_ref[...], (RB, PK * HID))
    for t in range(NT):
        acc = acc + jnp.dot(
            g_ref[t], w1e_ref[t], preferred_element_type=jnp.float32
        )
    h = jnp.maximum(acc, 0.0)
    raw = jnp.dot(h, w2e_ref[...], preferred_element_type=jnp.float32)
    out_ref[...] = jax.nn.sigmoid(raw + b2e_ref[...])


def _mlp(gathered, w1e, b1t, w2e, b2e):
    grid = (GR // RB,)
    return pl.pallas_call(
        _mlp_body,
        grid=grid,
        in_specs=[
            pl.BlockSpec((NT, RB, 128), lambda i: (0, i, 0)),
            pl.BlockSpec((NT, 128, PK * HID), lambda i: (0, 0, 0)),
            pl.BlockSpec((1, PK * HID), lambda i: (0, 0)),
            pl.BlockSpec((PK * HID, PK), lambda i: (0, 0)),
            pl.BlockSpec((1, PK), lambda i: (0, 0)),
        ],
        out_specs=pl.BlockSpec((RB, PK), lambda i: (i, 0)),
        out_shape=jax.ShapeDtypeStruct((GR, PK), jnp.float32),
    )(gathered, w1e, b1t, w2e, b2e)


def kernel(user_ids, item_ids, sparse_features, user_table, item_table,
           sparse_tables, W1, b1, W2, b2):
    uids = user_ids.astype(jnp.int32)
    tids = item_ids.astype(jnp.int32)
    sf = sparse_features.astype(jnp.int32)

    def _pack_idx(v):
        return (v // BLK) * BN + v % BN, ((v % BLK) // BN) * D

    iqu, imu = _pack_idx(uids)
    iqi, imi = _pack_idx(tids)
    iqs, ims = _pack_idx(sf)
    iqs = iqs + (jnp.arange(NS, dtype=jnp.int32) * FS)[None, :]
    iq_all = jnp.concatenate([iqu[:, None], iqi[:, None], iqs], axis=1)
    im_all = jnp.concatenate([imu[:, None], imi[:, None], ims], axis=1)

    def _worker_layout(a):
        a = a.reshape(NW, NCH, CHUNK, NT).transpose(0, 3, 1, 2)
        return a.reshape(NW, NQ, CHUNK)

    iq = _worker_layout(iq_all)
    im = _worker_layout(im_all)

    u4 = _tpack2(user_table.T)
    i4 = _tpack2(item_table.T)
    s4 = _tpack3(jnp.swapaxes(sparse_tables, 1, 2))

    gathered = _sc_gather(u4, i4, s4, iq, im)

    w1r = W1.reshape(NT, D, HID)
    w1e = jnp.zeros((NT, 128, PK * HID), jnp.float32)
    w2e = jnp.zeros((PK * HID, PK), jnp.float32)
    for k in range(PK):
        w1e = w1e.at[:, D * k:D * (k + 1), HID * k:HID * (k + 1)].set(w1r)
        w2e = w2e.at[HID * k:HID * (k + 1), k].set(W2[:, 0])
    b1t = jnp.tile(b1, PK).reshape(1, PK * HID)
    b2e = jnp.broadcast_to(b2.reshape(1, 1), (1, PK))

    out = _mlp(gathered, w1e, b1t, w2e, b2e)
    return out.reshape(B, 1)


# speedup vs baseline: 11.1440x; 1.1427x over previous
import functools

import jax
import jax.numpy as jnp
from jax import lax
from jax.experimental import pallas as pl
from jax.experimental.pallas import tpu as pltpu
from jax.experimental.pallas import tpu_sc as plsc

B = 16384
D = 32
NS = 26
SV = 100000
NT = NS + 2
HID = 2 * D

NC = 2
NSUB = 16
NW = NC * NSUB
BPW = B // NW
CHUNK = 128
NCH = BPW // CHUNK
NQ = NT * NCH

PK = 128 // D
GR = B * D // 128

UV = 1000000
BN = 2048
BLK = PK * BN
NBU = -(-UV // BLK)
NBS = -(-SV // BLK)
FS = NBS * BN


def _pack_t(xs):
    e = jnp.eye(128, dtype=jnp.bfloat16)
    acc = None
    for a, x in enumerate(xs):
        y = jax.lax.dot_general(
            x.astype(jnp.bfloat16), e[D * a:D * (a + 1), :],
            (((0,), (0,)), ((), ())),
            preferred_element_type=jnp.float32,
        )
        acc = y if acc is None else acc + y
    return acc


def _tpack2_body(x0, x1, x2, x3, o):
    o[...] = _pack_t([x0[...], x1[...], x2[...], x3[...]])


def _tpack2(tT):
    last = (UV - 1) // BN
    in_specs = [
        pl.BlockSpec((D, BN), lambda i, a=a: (0, jnp.minimum(PK * i + a, last)))
        for a in range(PK)
    ]
    return pl.pallas_call(
        _tpack2_body,
        grid=(NBU,),
        in_specs=in_specs,
        out_specs=pl.BlockSpec((BN, 128), lambda i: (i, 0)),
        out_shape=jax.ShapeDtypeStruct((NBU * BN, 128), jnp.float32),
    )(tT, tT, tT, tT)


def _tpack3_body(x0, x1, x2, x3, o):
    o[...] = _pack_t([x0[0], x1[0], x2[0], x3[0]])


def _tpack3(tT3):
    last = (SV - 1) // BN
    in_specs = [
        pl.BlockSpec(
            (1, D, BN),
            lambda f, i, a=a: (f, 0, jnp.minimum(PK * i + a, last)),
        )
        for a in range(PK)
    ]
    return pl.pallas_call(
        _tpack3_body,
        grid=(NS, NBS),
        in_specs=in_specs,
        out_specs=pl.BlockSpec((BN, 128), lambda f, i: (f * NBS + i, 0)),
        out_shape=jax.ShapeDtypeStruct((NS * FS, 128), jnp.float32),
    )(tT3, tT3, tT3, tT3)


def _make_sc_gather():
    mesh = plsc.VectorSubcoreMesh(core_axis_name="c", subcore_axis_name="s")

    @functools.partial(
        pl.kernel,
        mesh=mesh,
        out_type=jax.ShapeDtypeStruct((NT, GR, 128), jnp.float32),
        scratch_types=[
            pltpu.VMEM((NQ, 128), jnp.int32),
            pltpu.VMEM((NQ, 128), jnp.int32),
            pltpu.VMEM((BPW, 128), jnp.float32),
            pltpu.VMEM((128, 128), jnp.float32),
            pltpu.SemaphoreType.DMA,
        ],
        compiler_params=pltpu.CompilerParams(
            use_tc_tiling_on_sc=False, needs_layout_passes=False
        ),
    )
    def sc_gather(user_hbm, item_hbm, sparse_hbm, iq_hbm, im_hbm, out_hbm,
                  iqv, imv, rows_v, out_v, sem):
        wid = lax.axis_index("s") * NC + lax.axis_index("c")
        pltpu.sync_copy(iq_hbm.at[wid], iqv)
        pltpu.sync_copy(im_hbm.at[wid], imv)
        iota16 = lax.iota(jnp.int32, 16)

        def do_table(table, t):
            cps = [
                pltpu.async_copy(
                    table.at[iqv.at[t * NCH + c]],
                    rows_v.at[pl.ds(c * CHUNK, CHUNK)],
                    sem,
                )
                for c in range(NCH)
            ]
            for cp in cps:
                cp.wait()

            def grp_body(g, carry):
                r0 = g * 16
                m16 = imv.at[t * NCH + (r0 // CHUNK),
                             pl.ds(r0 % CHUNK, 16)][...]
                q0 = g * 4
                for j in range(16):
                    m = m16[j]
                    lo = rows_v.at[r0 + j, pl.ds(m, 16)][...]
                    hi = rows_v.at[r0 + j, pl.ds(m + 16, 16)][...]
                    out_v[q0 + j // 4, pl.ds((j % 4) * D, 16)] = lo
                    out_v[q0 + j // 4, pl.ds((j % 4) * D + 16, 16)] = hi
                return carry

            lax.fori_loop(0, BPW // 16, grp_body, 0)
            pltpu.sync_copy(out_v, out_hbm.at[t, pl.ds(wid * 128, 128)])

        do_table(user_hbm, 0)
        do_table(item_hbm, 1)

        def body(f, carry):
            do_table(sparse_hbm, f + 2)
            return carry

        lax.fori_loop(0, NS, body, 0)

    return sc_gather


_sc_gather = _make_sc_gather()


RB = 128


def _mlp_body(g_ref, w1e_ref, b1t_ref, w2e_ref, b2e_ref, out_ref):
    acc = jnp.broadcast_to(b1t_ref[...], (RB, PK * HID))
    for t in range(NT):
        acc = acc + jnp.dot(
            g_ref[t], w1e_ref[t], preferred_element_type=jnp.float32
        )
    h = jnp.maximum(acc, 0.0)
    raw = jnp.dot(h, w2e_ref[...], preferred_element_type=jnp.float32)
    out_ref[...] = jax.nn.sigmoid(raw + b2e_ref[...])


def _mlp(gathered, w1e, b1t, w2e, b2e):
    grid = (GR // RB,)
    return pl.pallas_call(
        _mlp_body,
        grid=grid,
        in_specs=[
            pl.BlockSpec((NT, RB, 128), lambda i: (0, i, 0)),
            pl.BlockSpec((NT, 128, PK * HID), lambda i: (0, 0, 0)),
            pl.BlockSpec((1, PK * HID), lambda i: (0, 0)),
            pl.BlockSpec((PK * HID, PK), lambda i: (0, 0)),
            pl.BlockSpec((1, PK), lambda i: (0, 0)),
        ],
        out_specs=pl.BlockSpec((RB, PK), lambda i: (i, 0)),
        out_shape=jax.ShapeDtypeStruct((GR, PK), jnp.float32),
    )(gathered, w1e, b1t, w2e, b2e)


def kernel(user_ids, item_ids, sparse_features, user_table, item_table,
           sparse_tables, W1, b1, W2, b2):
    uids = user_ids.astype(jnp.int32)
    tids = item_ids.astype(jnp.int32)
    sf = sparse_features.astype(jnp.int32)

    def _pack_idx(v):
        return (v // BLK) * BN + v % BN, ((v % BLK) // BN) * D

    iqu, imu = _pack_idx(uids)
    iqi, imi = _pack_idx(tids)
    iqs, ims = _pack_idx(sf)
    iqs = iqs + (jnp.arange(NS, dtype=jnp.int32) * FS)[None, :]
    iq_all = jnp.concatenate([iqu[:, None], iqi[:, None], iqs], axis=1)
    im_all = jnp.concatenate([imu[:, None], imi[:, None], ims], axis=1)

    def _worker_layout(a):
        a = a.reshape(NW, NCH, CHUNK, NT).transpose(0, 3, 1, 2)
        return a.reshape(NW, NQ, CHUNK)

    iq = _worker_layout(iq_all)
    im = _worker_layout(im_all)

    u4 = _tpack2(user_table.T)
    i4 = _tpack2(item_table.T)
    s4 = _tpack3(jnp.swapaxes(sparse_tables, 1, 2))

    gathered = _sc_gather(u4, i4, s4, iq, im)

    w1r = W1.reshape(NT, D, HID)
    w1e = jnp.zeros((NT, 128, PK * HID), jnp.float32)
    w2e = jnp.zeros((PK * HID, PK), jnp.float32)
    for k in range(PK):
        w1e = w1e.at[:, D * k:D * (k + 1), HID * k:HID * (k + 1)].set(w1r)
        w2e = w2e.at[HID * k:HID * (k + 1), k].set(W2[:, 0])
    b1t = jnp.tile(b1, PK).reshape(1, PK * HID)
    b2e = jnp.broadcast_to(b2.reshape(1, 1), (1, PK))

    out = _mlp(gathered, w1e, b1t, w2e, b2e)
    return out.reshape(B, 1)


# speedup vs baseline: 13.2805x; 1.1917x over previous
import functools

import jax
import jax.numpy as jnp
from jax import lax
from jax.experimental import pallas as pl
from jax.experimental.pallas import tpu as pltpu
from jax.experimental.pallas import tpu_sc as plsc

B = 16384
D = 32
NS = 26
SV = 100000
NT = NS + 2
HID = 2 * D

NC = 2
NSUB = 16
NW = NC * NSUB
BPW = B // NW
CHUNK = 128
NCH = BPW // CHUNK
NQ = NT * NCH

PK = 128 // D
GR = B * D // 128

UV = 1000000
BN = 4096
BLK = PK * BN
NBU = -(-UV // BLK)
NBS = -(-SV // BLK)
FS = NBS * BN


def _pack_t(xs):
    e = jnp.eye(128, dtype=jnp.bfloat16)
    acc = None
    for a, x in enumerate(xs):
        y = jax.lax.dot_general(
            x.astype(jnp.bfloat16), e[D * a:D * (a + 1), :],
            (((0,), (0,)), ((), ())),
            preferred_element_type=jnp.float32,
        )
        acc = y if acc is None else acc + y
    return acc


def _tpack2_body(x0, x1, x2, x3, o):
    o[...] = _pack_t([x0[...], x1[...], x2[...], x3[...]])


def _tpack2(tT):
    last = (UV - 1) // BN
    in_specs = [
        pl.BlockSpec((D, BN), lambda i, a=a: (0, jnp.minimum(PK * i + a, last)))
        for a in range(PK)
    ]
    return pl.pallas_call(
        _tpack2_body,
        grid=(NBU,),
        in_specs=in_specs,
        out_specs=pl.BlockSpec((BN, 128), lambda i: (i, 0)),
        out_shape=jax.ShapeDtypeStruct((NBU * BN, 128), jnp.float32),
    )(tT, tT, tT, tT)


def _tpack3_body(x0, x1, x2, x3, o):
    o[...] = _pack_t([x0[0], x1[0], x2[0], x3[0]])


def _tpack3(tT3):
    last = (SV - 1) // BN
    in_specs = [
        pl.BlockSpec(
            (1, D, BN),
            lambda f, i, a=a: (f, 0, jnp.minimum(PK * i + a, last)),
        )
        for a in range(PK)
    ]
    return pl.pallas_call(
        _tpack3_body,
        grid=(NS, NBS),
        in_specs=in_specs,
        out_specs=pl.BlockSpec((BN, 128), lambda f, i: (f * NBS + i, 0)),
        out_shape=jax.ShapeDtypeStruct((NS * FS, 128), jnp.float32),
    )(tT3, tT3, tT3, tT3)


def _make_sc_gather():
    mesh = plsc.VectorSubcoreMesh(core_axis_name="c", subcore_axis_name="s")

    @functools.partial(
        pl.kernel,
        mesh=mesh,
        out_type=jax.ShapeDtypeStruct((NT, GR, 128), jnp.float32),
        scratch_types=[
            pltpu.VMEM((NQ, 128), jnp.int32),
            pltpu.VMEM((NQ, 128), jnp.int32),
            pltpu.VMEM((BPW, 128), jnp.float32),
            pltpu.VMEM((128, 128), jnp.float32),
            pltpu.SemaphoreType.DMA,
        ],
        compiler_params=pltpu.CompilerParams(
            use_tc_tiling_on_sc=False, needs_layout_passes=False
        ),
    )
    def sc_gather(user_hbm, item_hbm, sparse_hbm, iq_hbm, im_hbm, out_hbm,
                  iqv, imv, rows_v, out_v, sem):
        wid = lax.axis_index("s") * NC + lax.axis_index("c")
        pltpu.sync_copy(iq_hbm.at[wid], iqv)
        pltpu.sync_copy(im_hbm.at[wid], imv)
        iota16 = lax.iota(jnp.int32, 16)

        def do_table(table, t):
            cps = [
                pltpu.async_copy(
                    table.at[iqv.at[t * NCH + c]],
                    rows_v.at[pl.ds(c * CHUNK, CHUNK)],
                    sem,
                )
                for c in range(NCH)
            ]
            for cp in cps:
                cp.wait()

            def grp_body(g, carry):
                r0 = g * 16
                m16 = imv.at[t * NCH + (r0 // CHUNK),
                             pl.ds(r0 % CHUNK, 16)][...]
                q0 = g * 4
                for j in range(16):
                    m = m16[j]
                    lo = rows_v.at[r0 + j, pl.ds(m, 16)][...]
                    hi = rows_v.at[r0 + j, pl.ds(m + 16, 16)][...]
                    out_v[q0 + j // 4, pl.ds((j % 4) * D, 16)] = lo
                    out_v[q0 + j // 4, pl.ds((j % 4) * D + 16, 16)] = hi
                return carry

            lax.fori_loop(0, BPW // 16, grp_body, 0)
            pltpu.sync_copy(out_v, out_hbm.at[t, pl.ds(wid * 128, 128)])

        do_table(user_hbm, 0)
        do_table(item_hbm, 1)

        def body(f, carry):
            do_table(sparse_hbm, f + 2)
            return carry

        lax.fori_loop(0, NS, body, 0)

    return sc_gather


_sc_gather = _make_sc_gather()


RB = 128


def _mlp_body(g_ref, w1e_ref, b1t_ref, w2e_ref, b2e_ref, out_ref):
    acc = jnp.broadcast_to(b1t_ref[...], (RB, PK * HID))
    for t in range(NT):
        acc = acc + jnp.dot(
            g_ref[t], w1e_ref[t], preferred_element_type=jnp.float32
        )
    h = jnp.maximum(acc, 0.0)
    raw = jnp.dot(h, w2e_ref[...], preferred_element_type=jnp.float32)
    out_ref[...] = jax.nn.sigmoid(raw + b2e_ref[...])


def _mlp(gathered, w1e, b1t, w2e, b2e):
    grid = (GR // RB,)
    return pl.pallas_call(
        _mlp_body,
        grid=grid,
        in_specs=[
            pl.BlockSpec((NT, RB, 128), lambda i: (0, i, 0)),
            pl.BlockSpec((NT, 128, PK * HID), lambda i: (0, 0, 0)),
            pl.BlockSpec((1, PK * HID), lambda i: (0, 0)),
            pl.BlockSpec((PK * HID, PK), lambda i: (0, 0)),
            pl.BlockSpec((1, PK), lambda i: (0, 0)),
        ],
        out_specs=pl.BlockSpec((RB, PK), lambda i: (i, 0)),
        out_shape=jax.ShapeDtypeStruct((GR, PK), jnp.float32),
    )(gathered, w1e, b1t, w2e, b2e)


def kernel(user_ids, item_ids, sparse_features, user_table, item_table,
           sparse_tables, W1, b1, W2, b2):
    uids = user_ids.astype(jnp.int32)
    tids = item_ids.astype(jnp.int32)
    sf = sparse_features.astype(jnp.int32)

    def _pack_idx(v):
        return (v // BLK) * BN + v % BN, ((v % BLK) // BN) * D

    iqu, imu = _pack_idx(uids)
    iqi, imi = _pack_idx(tids)
    iqs, ims = _pack_idx(sf.T)
    iqs = iqs + (jnp.arange(NS, dtype=jnp.int32) * FS)[:, None]
    iq_all = jnp.concatenate([iqu[None], iqi[None], iqs], axis=0)
    im_all = jnp.concatenate([imu[None], imi[None], ims], axis=0)

    def _worker_layout(a):
        a = a.reshape(NT, NW, NCH, CHUNK).transpose(1, 0, 2, 3)
        return a.reshape(NW, NQ, CHUNK)

    iq = _worker_layout(iq_all)
    im = _worker_layout(im_all)

    u4 = _tpack2(user_table.T)
    i4 = _tpack2(item_table.T)
    s4 = _tpack3(jnp.swapaxes(sparse_tables, 1, 2))

    gathered = _sc_gather(u4, i4, s4, iq, im)

    w1r = W1.reshape(NT, D, HID)
    w1e = jnp.zeros((NT, 128, PK * HID), jnp.float32)
    w2e = jnp.zeros((PK * HID, PK), jnp.float32)
    for k in range(PK):
        w1e = w1e.at[:, D * k:D * (k + 1), HID * k:HID * (k + 1)].set(w1r)
        w2e = w2e.at[HID * k:HID * (k + 1), k].set(W2[:, 0])
    b1t = jnp.tile(b1, PK).reshape(1, PK * HID)
    b2e = jnp.broadcast_to(b2.reshape(1, 1), (1, PK))

    out = _mlp(gathered, w1e, b1t, w2e, b2e)
    return out.reshape(B, 1)


# speedup vs baseline: 14.6156x; 1.1005x over previous
import functools

import jax
import jax.numpy as jnp
from jax import lax
from jax.experimental import pallas as pl
from jax.experimental.pallas import tpu as pltpu
from jax.experimental.pallas import tpu_sc as plsc

B = 16384
D = 32
NS = 26
SV = 100000
NT = NS + 2
HID = 2 * D

NC = 2
NSUB = 16
NW = NC * NSUB
BPW = B // NW
CHUNK = 128
NCH = BPW // CHUNK
NQ = NT * NCH

PK = 128 // D
GR = B * D // 128

UV = 1000000
BN = 4096
BLK = PK * BN
NBU = -(-UV // BLK)
NBS = -(-SV // BLK)
FS = NBS * BN


def _pack_t(xs):
    e = jnp.eye(128, dtype=jnp.bfloat16)
    acc = None
    for a, x in enumerate(xs):
        y = jax.lax.dot_general(
            x.astype(jnp.bfloat16), e[D * a:D * (a + 1), :],
            (((0,), (0,)), ((), ())),
            preferred_element_type=jnp.float32,
        )
        acc = y if acc is None else acc + y
    return acc


def _tpack2_body(x0, x1, x2, x3, o):
    o[...] = _pack_t([x0[...], x1[...], x2[...], x3[...]])


def _tpack2(tT):
    last = (UV - 1) // BN
    in_specs = [
        pl.BlockSpec((D, BN), lambda i, a=a: (0, jnp.minimum(PK * i + a, last)))
        for a in range(PK)
    ]
    return pl.pallas_call(
        _tpack2_body,
        grid=(NBU,),
        in_specs=in_specs,
        out_specs=pl.BlockSpec((BN, 128), lambda i: (i, 0)),
        out_shape=jax.ShapeDtypeStruct((NBU * BN, 128), jnp.float32),
    )(tT, tT, tT, tT)


def _tpack3_body(x0, x1, x2, x3, o):
    o[...] = _pack_t([x0[0], x1[0], x2[0], x3[0]])


def _tpack3(tT3):
    last = (SV - 1) // BN
    in_specs = [
        pl.BlockSpec(
            (1, D, BN),
            lambda f, i, a=a: (f, 0, jnp.minimum(PK * i + a, last)),
        )
        for a in range(PK)
    ]
    return pl.pallas_call(
        _tpack3_body,
        grid=(NS, NBS),
        in_specs=in_specs,
        out_specs=pl.BlockSpec((BN, 128), lambda f, i: (f * NBS + i, 0)),
        out_shape=jax.ShapeDtypeStruct((NS * FS, 128), jnp.float32),
    )(tT3, tT3, tT3, tT3)


def _sc_body(ntab, out_hbm, iqv, imv, rows_v, out_v, sem, wid, tables):
    def do_table(table, t):
        cps = [
            pltpu.async_copy(
                table.at[iqv.at[t * NCH + c]],
                rows_v.at[pl.ds(c * CHUNK, CHUNK)],
                sem,
            )
            for c in range(NCH)
        ]
        for cp in cps:
            cp.wait()

        def grp_body(g, carry):
            r0 = g * 16
            m16 = imv.at[t * NCH + (r0 // CHUNK),
                         pl.ds(r0 % CHUNK, 16)][...]
            q0 = g * 4
            for j in range(16):
                m = m16[j]
                lo = rows_v.at[r0 + j, pl.ds(m, 16)][...]
                hi = rows_v.at[r0 + j, pl.ds(m + 16, 16)][...]
                out_v[q0 + j // 4, pl.ds((j % 4) * D, 16)] = lo
                out_v[q0 + j // 4, pl.ds((j % 4) * D + 16, 16)] = hi
            return carry

        lax.fori_loop(0, BPW // 16, grp_body, 0)
        pltpu.sync_copy(out_v, out_hbm.at[t, pl.ds(wid * 128, 128)])

    if isinstance(tables, list):
        for table, t in tables:
            do_table(table, t)
    else:
        def body(f, carry):
            do_table(tables, f)
            return carry

        lax.fori_loop(0, ntab, body, 0)


def _sc_scratch(nq):
    return [
        pltpu.VMEM((nq, 128), jnp.int32),
        pltpu.VMEM((nq, 128), jnp.int32),
        pltpu.VMEM((BPW, 128), jnp.float32),
        pltpu.VMEM((128, 128), jnp.float32),
        pltpu.SemaphoreType.DMA,
    ]


_SC_PARAMS = pltpu.CompilerParams(
    use_tc_tiling_on_sc=False, needs_layout_passes=False
)


def _make_sc_gathers():
    mesh = plsc.VectorSubcoreMesh(core_axis_name="c", subcore_axis_name="s")

    @functools.partial(
        pl.kernel,
        mesh=mesh,
        out_type=jax.ShapeDtypeStruct((NS, GR, 128), jnp.float32),
        scratch_types=_sc_scratch(NS * NCH),
        compiler_params=_SC_PARAMS,
    )
    def sc_gather_sparse(sparse_hbm, iq_hbm, im_hbm, out_hbm,
                         iqv, imv, rows_v, out_v, sem):
        wid = lax.axis_index("s") * NC + lax.axis_index("c")
        pltpu.sync_copy(iq_hbm.at[wid], iqv)
        pltpu.sync_copy(im_hbm.at[wid], imv)
        _sc_body(NS, out_hbm, iqv, imv, rows_v, out_v, sem, wid, sparse_hbm)

    @functools.partial(
        pl.kernel,
        mesh=mesh,
        out_type=jax.ShapeDtypeStruct((2, GR, 128), jnp.float32),
        scratch_types=_sc_scratch(2 * NCH),
        compiler_params=_SC_PARAMS,
    )
    def sc_gather_ui(user_hbm, item_hbm, iq_hbm, im_hbm, out_hbm,
                     iqv, imv, rows_v, out_v, sem):
        wid = lax.axis_index("s") * NC + lax.axis_index("c")
        pltpu.sync_copy(iq_hbm.at[wid], iqv)
        pltpu.sync_copy(im_hbm.at[wid], imv)
        _sc_body(2, out_hbm, iqv, imv, rows_v, out_v, sem, wid,
                 [(user_hbm, 0), (item_hbm, 1)])

    return sc_gather_sparse, sc_gather_ui


_sc_gather_sparse, _sc_gather_ui = _make_sc_gathers()


RB = 128


def _mlp_body(gs_ref, gui_ref, w1e_ref, b1t_ref, w2e_ref, b2e_ref, out_ref):
    acc = jnp.broadcast_to(b1t_ref[...], (RB, PK * HID))
    for t in range(2):
        acc = acc + jnp.dot(
            gui_ref[t], w1e_ref[t], preferred_element_type=jnp.float32
        )
    for f in range(NS):
        acc = acc + jnp.dot(
            gs_ref[f], w1e_ref[2 + f], preferred_element_type=jnp.float32
        )
    h = jnp.maximum(acc, 0.0)
    raw = jnp.dot(h, w2e_ref[...], preferred_element_type=jnp.float32)
    out_ref[...] = jax.nn.sigmoid(raw + b2e_ref[...])


def _mlp(g_sparse, g_ui, w1e, b1t, w2e, b2e):
    grid = (GR // RB,)
    return pl.pallas_call(
        _mlp_body,
        grid=grid,
        in_specs=[
            pl.BlockSpec((NS, RB, 128), lambda i: (0, i, 0)),
            pl.BlockSpec((2, RB, 128), lambda i: (0, i, 0)),
            pl.BlockSpec((NT, 128, PK * HID), lambda i: (0, 0, 0)),
            pl.BlockSpec((1, PK * HID), lambda i: (0, 0)),
            pl.BlockSpec((PK * HID, PK), lambda i: (0, 0)),
            pl.BlockSpec((1, PK), lambda i: (0, 0)),
        ],
        out_specs=pl.BlockSpec((RB, PK), lambda i: (i, 0)),
        out_shape=jax.ShapeDtypeStruct((GR, PK), jnp.float32),
    )(g_sparse, g_ui, w1e, b1t, w2e, b2e)


def kernel(user_ids, item_ids, sparse_features, user_table, item_table,
           sparse_tables, W1, b1, W2, b2):
    uids = user_ids.astype(jnp.int32)
    tids = item_ids.astype(jnp.int32)
    sf = sparse_features.astype(jnp.int32)

    def _pack_idx(v):
        return (v // BLK) * BN + v % BN, ((v % BLK) // BN) * D

    iqu, imu = _pack_idx(uids)
    iqi, imi = _pack_idx(tids)
    iqs, ims = _pack_idx(sf.T)
    iqs = iqs + (jnp.arange(NS, dtype=jnp.int32) * FS)[:, None]

    def _worker_layout(a):
        nt = a.shape[0]
        a = a.reshape(nt, NW, NCH, CHUNK).transpose(1, 0, 2, 3)
        return a.reshape(NW, nt * NCH, CHUNK)

    iq_s = _worker_layout(iqs)
    im_s = _worker_layout(ims)
    iq_ui = _worker_layout(jnp.stack([iqu, iqi]))
    im_ui = _worker_layout(jnp.stack([imu, imi]))

    s4 = _tpack3(jnp.swapaxes(sparse_tables, 1, 2))
    g_sparse = _sc_gather_sparse(s4, iq_s, im_s)
    u4 = _tpack2(user_table.T)
    i4 = _tpack2(item_table.T)
    g_ui = _sc_gather_ui(u4, i4, iq_ui, im_ui)

    w1r = W1.reshape(NT, D, HID)
    w1e = jnp.zeros((NT, 128, PK * HID), jnp.float32)
    w2e = jnp.zeros((PK * HID, PK), jnp.float32)
    for k in range(PK):
        w1e = w1e.at[:, D * k:D * (k + 1), HID * k:HID * (k + 1)].set(w1r)
        w2e = w2e.at[HID * k:HID * (k + 1), k].set(W2[:, 0])
    b1t = jnp.tile(b1, PK).reshape(1, PK * HID)
    b2e = jnp.broadcast_to(b2.reshape(1, 1), (1, PK))

    out = _mlp(g_sparse, g_ui, w1e, b1t, w2e, b2e)
    return out.reshape(B, 1)
